# baseline (speedup 1.0000x reference)
import functools

import jax
import jax.numpy as jnp
from jax import lax
from jax.experimental import pallas as pl
from jax.experimental.pallas import tpu as pltpu
from jax.experimental.pallas import tpu_sc as plsc

NC, NS, L = 2, 16, 16
NW = NC * NS
CQ = 64
CHUNK = 128


def _make_sc_kernel(B, N, C, W, npix):
    qpb = C // CQ
    jobs_per_w = (B * qpb) // NW
    nchunk = N // CHUNK
    nv = CHUNK // L
    mesh = plsc.VectorSubcoreMesh(
        core_axis_name="c", subcore_axis_name="s", num_cores=NC, num_subcores=NS
    )

    @functools.partial(
        pl.kernel,
        out_type=jax.ShapeDtypeStruct((B, N, C), jnp.float32),
        mesh=mesh,
        compiler_params=pltpu.CompilerParams(
            needs_layout_passes=False, use_tc_tiling_on_sc=False
        ),
        scratch_types=[
            pltpu.VMEM((N * 2,), jnp.float32),
            pltpu.VMEM((N,), jnp.int32),
            pltpu.VMEM((CQ, npix), jnp.float32),
            pltpu.VMEM((CHUNK, CQ), jnp.float32),
            pltpu.SemaphoreType.DMA,
        ],
    )
    def sc_body(loc_hbm, fmap_hbm, out_hbm, loc_v, pix_v, planes_v, obuf, s0):
        cid = lax.axis_index("c")
        sid = lax.axis_index("s")
        wid = sid * NC + cid
        jid0 = wid * jobs_per_w
        b = jid0 // qpb
        q0 = jid0 % qpb

        pltpu.sync_copy(loc_hbm.at[b], loc_v)
        half = jnp.float32(W / 2.0)

        def idx_body(j, carry):
            xpos = lax.iota(jnp.int32, L) * 2 + j * (2 * L)
            x = plsc.load_gather(loc_v, [xpos])
            y = plsc.load_gather(loc_v, [xpos + 1])
            x = jnp.clip(x, -0.999, 0.999)
            y = jnp.clip(y, -0.999, 0.999)
            row = ((jnp.float32(1.0) - y) * half).astype(jnp.int32)
            col = ((jnp.float32(1.0) + x) * half).astype(jnp.int32)
            pix_v[pl.ds(j * L, L)] = row * W + col
            return carry

        lax.fori_loop(0, N // L, idx_body, 0, unroll=4)

        lane = lax.iota(jnp.int32, L)

        for k in range(jobs_per_w):
            q = q0 + k
            c0 = q * CQ
            pltpu.sync_copy(fmap_hbm.at[b, pl.ds(c0, CQ), :], planes_v)

            def chunk_body(ch, carry):
                n0 = ch * CHUNK
                for v in range(nv):
                    pix16 = pix_v[pl.ds(n0 + v * L, L)]
                    pt16 = lane + v * L
                    for c in range(CQ):
                        cc = jnp.full((L,), c, jnp.int32)
                        vals = plsc.load_gather(planes_v, [cc, pix16])
                        plsc.store_scatter(obuf, [pt16, cc], vals)
                pltpu.sync_copy(
                    obuf, out_hbm.at[b, pl.ds(n0, CHUNK), pl.ds(c0, CQ)]
                )
                return carry

            lax.fori_loop(0, nchunk, chunk_body, 0)

    return sc_body


def kernel(fmap, loc):
    B, C, H, W = fmap.shape
    N = loc.shape[1]
    npix = H * W
    sc = _make_sc_kernel(B, N, C, W, npix)
    return sc(loc.reshape(B, N * 2), fmap.reshape(B, C, npix))


# speedup vs baseline: 1.3866x; 1.3866x over previous
import functools

import jax
import jax.numpy as jnp
from jax import lax
from jax.experimental import pallas as pl
from jax.experimental.pallas import tpu as pltpu
from jax.experimental.pallas import tpu_sc as plsc

NC, NS, L = 2, 16, 16
NW = NC * NS
CQ = 64
CHUNK = 128


def _make_sc_kernel(B, N, C, W, npix):
    qpb = C // CQ
    jobs_per_w = (B * qpb) // NW
    nchunk = N // CHUNK
    nv = CHUNK // L
    mesh = plsc.VectorSubcoreMesh(
        core_axis_name="c", subcore_axis_name="s", num_cores=NC, num_subcores=NS
    )

    @functools.partial(
        pl.kernel,
        out_type=jax.ShapeDtypeStruct((B, N, C), jnp.float32),
        mesh=mesh,
        compiler_params=pltpu.CompilerParams(
            needs_layout_passes=False, use_tc_tiling_on_sc=False
        ),
        scratch_types=[
            pltpu.VMEM((N * 2,), jnp.float32),
            pltpu.VMEM((N,), jnp.int32),
            pltpu.VMEM((CQ, npix), jnp.float32),
            pltpu.VMEM((CHUNK, CQ), jnp.float32),
            pltpu.SemaphoreType.DMA,
        ],
    )
    def sc_body(loc_hbm, fmap_hbm, out_hbm, loc_v, pix_v, planes_v, obuf, s0):
        cid = lax.axis_index("c")
        sid = lax.axis_index("s")
        wid = sid * NC + cid
        jid0 = wid * jobs_per_w
        b = jid0 // qpb
        q0 = jid0 % qpb

        pltpu.sync_copy(loc_hbm.at[b], loc_v)
        half = jnp.float32(W / 2.0)

        lane = lax.iota(jnp.int32, L)

        @plsc.parallel_loop(0, N // L, unroll=4)
        def idx_body(j):
            xpos = lane * 2 + j * (2 * L)
            x = plsc.load_gather(loc_v, [xpos])
            y = plsc.load_gather(loc_v, [xpos + 1])
            x = jnp.clip(x, -0.999, 0.999)
            y = jnp.clip(y, -0.999, 0.999)
            row = ((jnp.float32(1.0) - y) * half).astype(jnp.int32)
            col = ((jnp.float32(1.0) + x) * half).astype(jnp.int32)
            pix_v[pl.ds(j * L, L)] = row * W + col

        for k in range(jobs_per_w):
            q = q0 + k
            c0 = q * CQ
            pltpu.sync_copy(fmap_hbm.at[b, pl.ds(c0, CQ), :], planes_v)

            def chunk_body(ch, carry):
                n0 = ch * CHUNK
                pixs = [pix_v[pl.ds(n0 + v * L, L)] for v in range(nv)]

                @plsc.parallel_loop(0, CQ, unroll=2)
                def gather_body(c):
                    cc = jnp.zeros((L,), jnp.int32) + c
                    for v in range(nv):
                        vals = plsc.load_gather(planes_v, [cc, pixs[v]])
                        plsc.store_scatter(obuf, [lane + v * L, cc], vals)

                pltpu.sync_copy(
                    obuf, out_hbm.at[b, pl.ds(n0, CHUNK), pl.ds(c0, CQ)]
                )
                return carry

            lax.fori_loop(0, nchunk, chunk_body, 0)

    return sc_body


def kernel(fmap, loc):
    B, C, H, W = fmap.shape
    N = loc.shape[1]
    npix = H * W
    sc = _make_sc_kernel(B, N, C, W, npix)
    return sc(loc.reshape(B, N * 2), fmap.reshape(B, C, npix))


# speedup vs baseline: 1.5813x; 1.1405x over previous
import functools

import jax
import jax.numpy as jnp
from jax import lax
from jax.experimental import pallas as pl
from jax.experimental.pallas import tpu as pltpu
from jax.experimental.pallas import tpu_sc as plsc

NC, NS, L = 2, 16, 16
NW = NC * NS
CQ = 64
CHUNK = 128


def _make_sc_kernel(B, N, C, H, W):
    qpb = C // CQ
    jobs_per_w = (B * qpb) // NW
    nchunk = N // CHUNK
    nv = CHUNK // L
    wshift = W.bit_length() - 1
    mesh = plsc.VectorSubcoreMesh(
        core_axis_name="c", subcore_axis_name="s", num_cores=NC, num_subcores=NS
    )

    @functools.partial(
        pl.kernel,
        out_type=jax.ShapeDtypeStruct((B, N, C), jnp.float32),
        mesh=mesh,
        compiler_params=pltpu.CompilerParams(
            needs_layout_passes=False, use_tc_tiling_on_sc=False
        ),
        scratch_types=[
            pltpu.VMEM((N, 2), jnp.float32),
            pltpu.VMEM((N,), jnp.int32),
            pltpu.VMEM((CQ, H, W), jnp.float32),
            pltpu.VMEM((CHUNK, CQ + 1), jnp.float32),
            pltpu.SemaphoreType.DMA,
        ],
    )
    def sc_body(loc_hbm, fmap_hbm, out_hbm, loc_v, pix_v, planes_v, obuf, s0):
        cid = lax.axis_index("c")
        sid = lax.axis_index("s")
        wid = sid * NC + cid
        jid0 = wid * jobs_per_w
        b = jid0 // qpb
        q0 = jid0 % qpb

        pltpu.sync_copy(loc_hbm.at[b], loc_v)
        half = jnp.float32(W / 2.0)
        lane = lax.iota(jnp.int32, L)

        @plsc.parallel_loop(0, N // L, unroll=4)
        def idx_body(j):
            pids = lane + j * L
            zero = jnp.zeros((L,), jnp.int32)
            x = plsc.load_gather(loc_v, [pids, zero])
            y = plsc.load_gather(loc_v, [pids, zero + 1])
            x = jnp.clip(x, -0.999, 0.999)
            y = jnp.clip(y, -0.999, 0.999)
            row = ((jnp.float32(1.0) - y) * half).astype(jnp.int32)
            col = ((jnp.float32(1.0) + x) * half).astype(jnp.int32)
            pix_v[pl.ds(j * L, L)] = row * W + col

        for k in range(jobs_per_w):
            q = q0 + k
            c0 = q * CQ
            pltpu.sync_copy(fmap_hbm.at[b, pl.ds(c0, CQ)], planes_v)

            def chunk_body(ch, carry):
                n0 = ch * CHUNK
                pixs = [pix_v[pl.ds(n0 + v * L, L)] for v in range(nv)]
                rows = [lax.shift_right_logical(p, wshift) for p in pixs]
                cols = [jnp.bitwise_and(p, W - 1) for p in pixs]

                @plsc.parallel_loop(0, CQ, unroll=2)
                def gather_body(c):
                    cc = jnp.zeros((L,), jnp.int32) + c
                    for v in range(nv):
                        vals = plsc.load_gather(planes_v, [cc, rows[v], cols[v]])
                        plsc.store_scatter(obuf, [lane + v * L, cc], vals)

                pltpu.sync_copy(
                    obuf.at[:, pl.ds(0, CQ)],
                    out_hbm.at[b, pl.ds(n0, CHUNK), pl.ds(c0, CQ)],
                )
                return carry

            lax.fori_loop(0, nchunk, chunk_body, 0)

    return sc_body


def kernel(fmap, loc):
    B, C, H, W = fmap.shape
    N = loc.shape[1]
    sc = _make_sc_kernel(B, N, C, H, W)
    return sc(loc, fmap)


# speedup vs baseline: 2.2797x; 1.4416x over previous
import functools

import jax
import jax.numpy as jnp
from jax import lax
from jax.experimental import pallas as pl
from jax.experimental.pallas import tpu as pltpu
from jax.experimental.pallas import tpu_sc as plsc

NC, NS, L = 2, 16, 16
NW = NC * NS
CHUNK = 64


def _tr_body(f_ref, o_ref):
    c = f_ref.shape[1]
    npix = o_ref.shape[1]
    o_ref[0] = jnp.transpose(f_ref[0].reshape(c, npix), (1, 0))


def _transpose(fmap):
    b, c, h, w = fmap.shape
    npix = h * w
    return pl.pallas_call(
        _tr_body,
        grid=(b,),
        in_specs=[pl.BlockSpec((1, c, h, w), lambda i: (i, 0, 0, 0))],
        out_specs=pl.BlockSpec((1, npix, c), lambda i: (i, 0, 0)),
        out_shape=jax.ShapeDtypeStruct((b, npix, c), fmap.dtype),
    )(fmap)


def _make_sc_gather(B, N, C, W, npix):
    pts_per_w = (B * N) // NW
    w_per_b = N // pts_per_w
    nchunk = pts_per_w // CHUNK
    mesh = plsc.VectorSubcoreMesh(
        core_axis_name="c", subcore_axis_name="s", num_cores=NC, num_subcores=NS
    )

    @functools.partial(
        pl.kernel,
        out_type=jax.ShapeDtypeStruct((B, N, C), jnp.float32),
        mesh=mesh,
        compiler_params=pltpu.CompilerParams(needs_layout_passes=False),
        scratch_types=[
            pltpu.VMEM((pts_per_w,), jnp.float32),
            pltpu.VMEM((pts_per_w,), jnp.float32),
            pltpu.VMEM((nchunk, CHUNK), jnp.int32),
            pltpu.VMEM((CHUNK, C), jnp.float32),
            pltpu.VMEM((CHUNK, C), jnp.float32),
            pltpu.SemaphoreType.DMA,
            pltpu.SemaphoreType.DMA,
            pltpu.SemaphoreType.DMA,
            pltpu.SemaphoreType.DMA,
        ],
    )
    def sc_gather(x_hbm, y_hbm, table_hbm, out_hbm, x_v, y_v, idx_v, r0, r1, g0, g1, o0, o1):
        cid = lax.axis_index("c")
        sid = lax.axis_index("s")
        wid = sid * NC + cid
        b = wid // w_per_b
        noff = (wid % w_per_b) * pts_per_w

        pltpu.sync_copy(x_hbm.at[b, pl.ds(noff, pts_per_w)], x_v)
        pltpu.sync_copy(y_hbm.at[b, pl.ds(noff, pts_per_w)], y_v)

        half = jnp.float32(W / 2.0)
        lane = lax.iota(jnp.int32, L)
        base = b * npix

        @plsc.parallel_loop(0, pts_per_w // L, unroll=4)
        def idx_body(j):
            x = x_v[pl.ds(j * L, L)]
            y = y_v[pl.ds(j * L, L)]
            x = jnp.clip(x, -0.999, 0.999)
            y = jnp.clip(y, -0.999, 0.999)
            row = ((jnp.float32(1.0) - y) * half).astype(jnp.int32)
            col = ((jnp.float32(1.0) + x) * half).astype(jnp.int32)
            vpc = CHUNK // L
            idx_v[j // vpc, pl.ds((j % vpc) * L, L)] = row * W + col + base

        rows = [r0, r1]
        gsem = [g0, g1]
        osem = [o0, o1]

        def start_gather(ch, par):
            cp = pltpu.make_async_copy(
                table_hbm.at[idx_v.at[ch]], rows[par], gsem[par]
            )
            cp.start()
            return cp

        def start_out(ch, par):
            cp = pltpu.make_async_copy(
                rows[par], out_hbm.at[b, pl.ds(noff + ch * CHUNK, CHUNK), :], osem[par]
            )
            cp.start()
            return cp

        gcp = [start_gather(0, 0), start_gather(1, 1)]
        ocp = [None, None]
        for ch in range(nchunk):
            par = ch % 2
            gcp[par].wait()
            ocp[par] = start_out(ch, par)
            if ch + 2 < nchunk:
                ocp[par].wait()
                gcp[par] = start_gather(ch + 2, par)
        ocp[(nchunk - 2) % 2].wait()
        ocp[(nchunk - 1) % 2].wait()

    return sc_gather


def kernel(fmap, loc):
    B, C, H, W = fmap.shape
    N = loc.shape[1]
    npix = H * W
    table = _transpose(fmap).reshape(B * npix, C)
    sc_gather = _make_sc_gather(B, N, C, W, npix)
    return sc_gather(loc[..., 0], loc[..., 1], table)


# speedup vs baseline: 4.0534x; 1.7780x over previous
import functools

import jax
import jax.numpy as jnp
from jax import lax
from jax.experimental import pallas as pl
from jax.experimental.pallas import tpu as pltpu
from jax.experimental.pallas import tpu_sc as plsc

NC, NS, L = 2, 16, 16
NW = NC * NS
CHUNK = 64


def _tr_body(f_ref, o_ref):
    o_ref[0] = jnp.transpose(f_ref[0], (1, 0))


def _transpose(f3):
    b, c, npix = f3.shape
    return pl.pallas_call(
        _tr_body,
        grid=(b,),
        in_specs=[pl.BlockSpec((1, c, npix), lambda i: (i, 0, 0))],
        out_specs=pl.BlockSpec((1, npix, c), lambda i: (i, 0, 0)),
        out_shape=jax.ShapeDtypeStruct((b, npix, c), f3.dtype),
    )(f3)


def _make_sc_gather(B, N, C, W, npix):
    pts_per_w = (B * N) // NW
    w_per_b = N // pts_per_w
    nchunk = pts_per_w // CHUNK
    mesh = plsc.VectorSubcoreMesh(
        core_axis_name="c", subcore_axis_name="s", num_cores=NC, num_subcores=NS
    )

    @functools.partial(
        pl.kernel,
        out_type=jax.ShapeDtypeStruct((B, N, C), jnp.float32),
        mesh=mesh,
        compiler_params=pltpu.CompilerParams(needs_layout_passes=False),
        scratch_types=[
            pltpu.VMEM((pts_per_w,), jnp.float32),
            pltpu.VMEM((pts_per_w,), jnp.float32),
            pltpu.VMEM((nchunk, CHUNK), jnp.int32),
            pltpu.VMEM((CHUNK, C), jnp.float32),
            pltpu.VMEM((CHUNK, C), jnp.float32),
            pltpu.SemaphoreType.DMA,
            pltpu.SemaphoreType.DMA,
            pltpu.SemaphoreType.DMA,
            pltpu.SemaphoreType.DMA,
        ],
    )
    def sc_gather(x_hbm, y_hbm, table_hbm, out_hbm, x_v, y_v, idx_v, r0, r1, g0, g1, o0, o1):
        cid = lax.axis_index("c")
        sid = lax.axis_index("s")
        wid = sid * NC + cid
        b = wid // w_per_b
        noff = (wid % w_per_b) * pts_per_w

        pltpu.sync_copy(x_hbm.at[b, pl.ds(noff, pts_per_w)], x_v)
        pltpu.sync_copy(y_hbm.at[b, pl.ds(noff, pts_per_w)], y_v)

        half = jnp.float32(W / 2.0)
        lane = lax.iota(jnp.int32, L)
        base = b * npix

        @plsc.parallel_loop(0, pts_per_w // L, unroll=4)
        def idx_body(j):
            x = x_v[pl.ds(j * L, L)]
            y = y_v[pl.ds(j * L, L)]
            x = jnp.clip(x, -0.999, 0.999)
            y = jnp.clip(y, -0.999, 0.999)
            row = ((jnp.float32(1.0) - y) * half).astype(jnp.int32)
            col = ((jnp.float32(1.0) + x) * half).astype(jnp.int32)
            vpc = CHUNK // L
            idx_v[j // vpc, pl.ds((j % vpc) * L, L)] = row * W + col + base

        rows = [r0, r1]
        gsem = [g0, g1]
        osem = [o0, o1]

        def start_gather(ch, par):
            cp = pltpu.make_async_copy(
                table_hbm.at[idx_v.at[ch]], rows[par], gsem[par]
            )
            cp.start()
            return cp

        def start_out(ch, par):
            cp = pltpu.make_async_copy(
                rows[par], out_hbm.at[b, pl.ds(noff + ch * CHUNK, CHUNK), :], osem[par]
            )
            cp.start()
            return cp

        gcp = [start_gather(0, 0), start_gather(1, 1)]
        ocp = [None, None]
        for ch in range(nchunk):
            par = ch % 2
            gcp[par].wait()
            ocp[par] = start_out(ch, par)
            if ch + 2 < nchunk:
                ocp[par].wait()
                gcp[par] = start_gather(ch + 2, par)
        ocp[(nchunk - 2) % 2].wait()
        ocp[(nchunk - 1) % 2].wait()

    return sc_gather


def kernel(fmap, loc):
    B, C, H, W = fmap.shape
    N = loc.shape[1]
    npix = H * W
    table = _transpose(fmap.reshape(B, C, npix)).reshape(B * npix, C)
    sc_gather = _make_sc_gather(B, N, C, W, npix)
    return sc_gather(loc[..., 0], loc[..., 1], table)


# speedup vs baseline: 6.5805x; 1.6235x over previous
import functools

import jax
import jax.numpy as jnp
from jax import lax
from jax.experimental import pallas as pl
from jax.experimental.pallas import tpu as pltpu
from jax.experimental.pallas import tpu_sc as plsc

NC, NS, L = 2, 16, 16
NW = NC * NS
CHUNK = 64


def _make_sc_gather(B, N, C, W, npix):
    pts_per_w = (B * N) // NW
    w_per_b = N // pts_per_w
    nchunk = pts_per_w // CHUNK
    mesh = plsc.VectorSubcoreMesh(
        core_axis_name="c", subcore_axis_name="s", num_cores=NC, num_subcores=NS
    )

    @functools.partial(
        pl.kernel,
        out_type=jax.ShapeDtypeStruct((B, N, C), jnp.float32),
        mesh=mesh,
        compiler_params=pltpu.CompilerParams(needs_layout_passes=False),
        scratch_types=[
            pltpu.VMEM((pts_per_w,), jnp.float32),
            pltpu.VMEM((pts_per_w,), jnp.float32),
            pltpu.VMEM((nchunk, CHUNK), jnp.int32),
            pltpu.VMEM((CHUNK, C), jnp.float32),
            pltpu.VMEM((CHUNK, C), jnp.float32),
            pltpu.SemaphoreType.DMA,
            pltpu.SemaphoreType.DMA,
            pltpu.SemaphoreType.DMA,
            pltpu.SemaphoreType.DMA,
        ],
    )
    def sc_gather(x_hbm, y_hbm, table_hbm, out_hbm, x_v, y_v, idx_v, r0, r1, g0, g1, o0, o1):
        cid = lax.axis_index("c")
        sid = lax.axis_index("s")
        wid = sid * NC + cid
        b = wid // w_per_b
        noff = (wid % w_per_b) * pts_per_w

        pltpu.sync_copy(x_hbm.at[b, pl.ds(noff, pts_per_w)], x_v)
        pltpu.sync_copy(y_hbm.at[b, pl.ds(noff, pts_per_w)], y_v)

        half = jnp.float32(W / 2.0)
        lane = lax.iota(jnp.int32, L)
        base = b * npix

        @plsc.parallel_loop(0, pts_per_w // L, unroll=4)
        def idx_body(j):
            x = x_v[pl.ds(j * L, L)]
            y = y_v[pl.ds(j * L, L)]
            x = jnp.clip(x, -0.999, 0.999)
            y = jnp.clip(y, -0.999, 0.999)
            row = ((jnp.float32(1.0) - y) * half).astype(jnp.int32)
            col = ((jnp.float32(1.0) + x) * half).astype(jnp.int32)
            vpc = CHUNK // L
            idx_v[j // vpc, pl.ds((j % vpc) * L, L)] = row * W + col + base

        rows = [r0, r1]
        gsem = [g0, g1]
        osem = [o0, o1]

        def start_gather(ch, par):
            cp = pltpu.make_async_copy(
                table_hbm.at[idx_v.at[ch]], rows[par], gsem[par]
            )
            cp.start()
            return cp

        def start_out(ch, par):
            cp = pltpu.make_async_copy(
                rows[par], out_hbm.at[b, pl.ds(noff + ch * CHUNK, CHUNK), :], osem[par]
            )
            cp.start()
            return cp

        gcp = [start_gather(0, 0), start_gather(1, 1)]
        ocp = [None, None]
        for ch in range(nchunk):
            par = ch % 2
            gcp[par].wait()
            ocp[par] = start_out(ch, par)
            if ch + 2 < nchunk:
                ocp[par].wait()
                gcp[par] = start_gather(ch + 2, par)
        ocp[(nchunk - 2) % 2].wait()
        ocp[(nchunk - 1) % 2].wait()

    return sc_gather


def kernel(fmap, loc):
    B, C, H, W = fmap.shape
    N = loc.shape[1]
    npix = H * W
    table = jnp.transpose(fmap, (0, 2, 3, 1)).reshape(B * npix, C)
    sc_gather = _make_sc_gather(B, N, C, W, npix)
    return sc_gather(loc[..., 0], loc[..., 1], table)


# speedup vs baseline: 6.5885x; 1.0012x over previous
import functools

import jax
import jax.numpy as jnp
from jax import lax
from jax.experimental import pallas as pl
from jax.experimental.pallas import tpu as pltpu
from jax.experimental.pallas import tpu_sc as plsc

NC, NS, L = 2, 16, 16
NW = NC * NS
CHUNK = 32
NBUF = 4


def _make_sc_gather(B, N, C, W, npix):
    pts_per_w = (B * N) // NW
    w_per_b = N // pts_per_w
    nchunk = pts_per_w // CHUNK
    mesh = plsc.VectorSubcoreMesh(
        core_axis_name="c", subcore_axis_name="s", num_cores=NC, num_subcores=NS
    )

    @functools.partial(
        pl.kernel,
        out_type=jax.ShapeDtypeStruct((B, N, C), jnp.float32),
        mesh=mesh,
        compiler_params=pltpu.CompilerParams(needs_layout_passes=False),
        scratch_types=[
            pltpu.VMEM((pts_per_w,), jnp.float32),
            pltpu.VMEM((pts_per_w,), jnp.float32),
            pltpu.VMEM((nchunk, CHUNK), jnp.int32),
            [pltpu.VMEM((CHUNK, C), jnp.float32) for _ in range(NBUF)],
            [pltpu.SemaphoreType.DMA for _ in range(NBUF)],
            [pltpu.SemaphoreType.DMA for _ in range(NBUF)],
        ],
    )
    def sc_gather(x_hbm, y_hbm, table_hbm, out_hbm, x_v, y_v, idx_v, rows, gsem, osem):
        cid = lax.axis_index("c")
        sid = lax.axis_index("s")
        wid = sid * NC + cid
        b = wid // w_per_b
        noff = (wid % w_per_b) * pts_per_w

        pltpu.sync_copy(x_hbm.at[b, pl.ds(noff, pts_per_w)], x_v)
        pltpu.sync_copy(y_hbm.at[b, pl.ds(noff, pts_per_w)], y_v)

        half = jnp.float32(W / 2.0)
        lane = lax.iota(jnp.int32, L)
        base = b * npix

        @plsc.parallel_loop(0, pts_per_w // L, unroll=4)
        def idx_body(j):
            x = x_v[pl.ds(j * L, L)]
            y = y_v[pl.ds(j * L, L)]
            x = jnp.clip(x, -0.999, 0.999)
            y = jnp.clip(y, -0.999, 0.999)
            row = ((jnp.float32(1.0) - y) * half).astype(jnp.int32)
            col = ((jnp.float32(1.0) + x) * half).astype(jnp.int32)
            vpc = CHUNK // L
            idx_v[j // vpc, pl.ds((j % vpc) * L, L)] = row * W + col + base

        def start_gather(ch, par):
            cp = pltpu.make_async_copy(
                table_hbm.at[idx_v.at[ch]], rows[par], gsem[par]
            )
            cp.start()
            return cp

        def start_out(ch, par):
            cp = pltpu.make_async_copy(
                rows[par],
                out_hbm.at[b, pl.ds(noff + ch * CHUNK, CHUNK), :],
                osem[par],
            )
            cp.start()
            return cp

        gcp = [start_gather(ch, ch) for ch in range(NBUF)]
        ocp = [None] * NBUF
        for ch in range(nchunk):
            par = ch % NBUF
            gcp[par].wait()
            ocp[par] = start_out(ch, par)
            if ch + NBUF < nchunk:
                ocp[par].wait()
                gcp[par] = start_gather(ch + NBUF, par)
        for ch in range(nchunk - NBUF, nchunk):
            ocp[ch % NBUF].wait()

    return sc_gather


def kernel(fmap, loc):
    B, C, H, W = fmap.shape
    N = loc.shape[1]
    npix = H * W
    table = jnp.transpose(fmap, (0, 2, 3, 1)).reshape(B * npix, C)
    sc_gather = _make_sc_gather(B, N, C, W, npix)
    return sc_gather(loc[..., 0], loc[..., 1], table)


# speedup vs baseline: 6.6294x; 1.0062x over previous
import functools

import jax
import jax.numpy as jnp
from jax import lax
from jax.experimental import pallas as pl
from jax.experimental.pallas import tpu as pltpu
from jax.experimental.pallas import tpu_sc as plsc

NC, NS, L = 2, 16, 16
NW = NC * NS
CHUNK = 32
NBUF = 4


def _make_sc_gather(B, N, C, W, npix):
    pts_per_w = (B * N) // NW
    w_per_b = N // pts_per_w
    nchunk = pts_per_w // CHUNK
    mesh = plsc.VectorSubcoreMesh(
        core_axis_name="c", subcore_axis_name="s", num_cores=NC, num_subcores=NS
    )

    @functools.partial(
        pl.kernel,
        out_type=jax.ShapeDtypeStruct((B, N, C), jnp.float32),
        mesh=mesh,
        compiler_params=pltpu.CompilerParams(needs_layout_passes=False),
        scratch_types=[
            pltpu.VMEM((pts_per_w,), jnp.float32),
            pltpu.VMEM((pts_per_w,), jnp.float32),
            pltpu.VMEM((nchunk, CHUNK), jnp.int32),
            [pltpu.VMEM((CHUNK, C), jnp.float32) for _ in range(NBUF)],
            [pltpu.SemaphoreType.DMA for _ in range(NBUF)],
            [pltpu.SemaphoreType.DMA for _ in range(NBUF)],
        ],
    )
    def sc_gather(loc_t_hbm, table_hbm, out_hbm, x_v, y_v, idx_v, rows, gsem, osem):
        cid = lax.axis_index("c")
        sid = lax.axis_index("s")
        wid = sid * NC + cid
        b = wid // w_per_b
        noff = (wid % w_per_b) * pts_per_w

        pltpu.sync_copy(loc_t_hbm.at[b, 0, pl.ds(noff, pts_per_w)], x_v)
        pltpu.sync_copy(loc_t_hbm.at[b, 1, pl.ds(noff, pts_per_w)], y_v)

        half = jnp.float32(W / 2.0)
        lane = lax.iota(jnp.int32, L)
        base = b * npix

        @plsc.parallel_loop(0, pts_per_w // L, unroll=4)
        def idx_body(j):
            x = x_v[pl.ds(j * L, L)]
            y = y_v[pl.ds(j * L, L)]
            x = jnp.clip(x, -0.999, 0.999)
            y = jnp.clip(y, -0.999, 0.999)
            row = ((jnp.float32(1.0) - y) * half).astype(jnp.int32)
            col = ((jnp.float32(1.0) + x) * half).astype(jnp.int32)
            vpc = CHUNK // L
            idx_v[j // vpc, pl.ds((j % vpc) * L, L)] = row * W + col + base

        def start_gather(ch, par):
            cp = pltpu.make_async_copy(
                table_hbm.at[idx_v.at[ch]], rows[par], gsem[par]
            )
            cp.start()
            return cp

        def start_out(ch, par):
            cp = pltpu.make_async_copy(
                rows[par],
                out_hbm.at[b, pl.ds(noff + ch * CHUNK, CHUNK), :],
                osem[par],
            )
            cp.start()
            return cp

        gcp = [start_gather(ch, ch) for ch in range(NBUF)]
        ocp = [None] * NBUF
        for ch in range(nchunk):
            par = ch % NBUF
            gcp[par].wait()
            ocp[par] = start_out(ch, par)
            if ch + NBUF < nchunk:
                ocp[par].wait()
                gcp[par] = start_gather(ch + NBUF, par)
        for ch in range(nchunk - NBUF, nchunk):
            ocp[ch % NBUF].wait()

    return sc_gather


def kernel(fmap, loc):
    B, C, H, W = fmap.shape
    N = loc.shape[1]
    npix = H * W
    table = jnp.transpose(fmap, (0, 2, 3, 1)).reshape(B * npix, C)
    sc_gather = _make_sc_gather(B, N, C, W, npix)
    return sc_gather(jnp.transpose(loc, (0, 2, 1)), table)
